# baseline (speedup 1.0000x reference)
import functools

import jax
import jax.numpy as jnp
from jax import lax
from jax.experimental import pallas as pl
from jax.experimental.pallas import tpu as pltpu
from jax.experimental.pallas import tpu_sc as plsc

RADIUS = 0.2
K = 64
NC = 2
NS = 16
L = 16


def _cand_body(nxyz_t_ref, xyz_ref, cand_ref):
    nx = nxyz_t_ref[0]
    p = xyz_ref[0]
    qx = nx[:, 0:1]
    qy = nx[:, 1:2]
    qz = nx[:, 2:3]
    px = p[0:1, :]
    py = p[1:2, :]
    pz = p[2:3, :]
    cross = jax.lax.dot_general(nx, p, (((1,), (0,)), ((), ())),
                                preferred_element_type=jnp.float32)
    q2 = (qx * qx + qy * qy) + qz * qz
    p2 = (px * px + py * py) + pz * pz
    d2 = (q2 + p2) - 2.0 * cross
    iota = jax.lax.broadcasted_iota(jnp.int32, d2.shape, 1)
    r2 = jnp.float32(RADIUS * RADIUS)
    cand_ref[0] = jnp.where(d2 < r2, iota, -1)


def _ball_query_cand(new_xyz, xyz):
    B, _, M = new_xyz.shape
    N = xyz.shape[2]
    Mb = 256
    nxyz_t = jnp.transpose(new_xyz, (0, 2, 1))
    grid = (B, M // Mb)
    return pl.pallas_call(
        _cand_body,
        grid=grid,
        in_specs=[
            pl.BlockSpec((1, Mb, 3), lambda b, i: (b, i, 0)),
            pl.BlockSpec((1, 3, N), lambda b, i: (b, 0, 0)),
        ],
        out_specs=pl.BlockSpec((1, Mb, N), lambda b, i: (b, i, 0)),
        out_shape=jax.ShapeDtypeStruct((B, M, N), jnp.int32),
    )(nxyz_t, xyz)


def _make_select(B, M, n):
    rows = B * M
    ngroups = n // L
    rpw = rows // (NC * NS)
    npairs = rpw // 2
    UNROLL = 8
    mesh = plsc.VectorSubcoreMesh(core_axis_name="c", subcore_axis_name="s")

    @functools.partial(
        pl.kernel,
        out_type=jax.ShapeDtypeStruct((rows * K,), jnp.int32),
        mesh=mesh,
        compiler_params=pltpu.CompilerParams(
            needs_layout_passes=False, use_tc_tiling_on_sc=True),
        scratch_types=[
            pltpu.VMEM((n,), jnp.int32),
            pltpu.VMEM((n,), jnp.int32),
            pltpu.VMEM((n + L,), jnp.int32),
            pltpu.VMEM((rpw * K,), jnp.int32),
            pltpu.SemaphoreType.DMA,
            pltpu.SemaphoreType.DMA,
        ],
    )
    def select(cand_hbm, idx_hbm, cbuf0, cbuf1, rowbuf, outbuf, sem0, sem1):
        cid = lax.axis_index("c")
        sid = lax.axis_index("s")
        wid = sid * NC + cid
        base = wid * rpw

        zeros16 = jnp.zeros((L,), jnp.int32)

        def start_row_copy(r, cbuf, sem):
            pltpu.async_copy(cand_hbm.at[r // M, r % M], cbuf, sem)

        def wait_row_copy(r, cbuf, sem):
            pltpu.make_async_copy(cand_hbm.at[r // M, r % M], cbuf, sem).wait()

        BLK = 16

        def scan_row(cbuf, r_local):
            def body(w, ptrv):
                v = cbuf[pl.ds(w * L, L)]
                msk = v >= 0
                cum = plsc.cumsum(msk.astype(jnp.int32))
                pos = ptrv + cum - 1
                mske = jnp.logical_and(msk, pos < K)
                plsc.store_scatter(rowbuf, [pos], v, mask=mske)
                return ptrv + plsc.all_reduce_population_count(msk)

            def blk(bi, ptrv):
                return lax.cond(
                    ptrv[0] >= K,
                    lambda p: p,
                    lambda p: plsc.parallel_loop(
                        bi * BLK, (bi + 1) * BLK, 1, unroll=UNROLL, carry=p)(body),
                    ptrv)

            ptrv = lax.fori_loop(0, ngroups // BLK, blk, zeros16)
            v0 = rowbuf[pl.ds(0, L)]
            lanes0 = lax.iota(jnp.int32, L)
            firstv = plsc.cummax(jnp.where(lanes0 == 0, v0, jnp.int32(-2147483648)))
            fvec = jnp.where(ptrv > 0, firstv, 0)
            obase = r_local * K
            for g in range(K // L):
                cur = rowbuf[pl.ds(g * L, L)]
                lanes = lax.iota(jnp.int32, L) + (g * L)
                outbuf[pl.ds(obase + g * L, L)] = jnp.where(lanes < ptrv, cur, fvec)

        start_row_copy(base, cbuf0, sem0)

        def pair(j, _):
            r0 = base + 2 * j
            start_row_copy(r0 + 1, cbuf1, sem1)
            wait_row_copy(r0, cbuf0, sem0)
            scan_row(cbuf0, 2 * j)

            @pl.when(j < npairs - 1)
            def _():
                start_row_copy(r0 + 2, cbuf0, sem0)

            wait_row_copy(r0 + 1, cbuf1, sem1)
            scan_row(cbuf1, 2 * j + 1)
            return 0

        lax.fori_loop(0, npairs, pair, 0)
        pltpu.sync_copy(outbuf, idx_hbm.at[pl.ds(base * K, rpw * K)])

    return select


def _make_gather(B, C, M, N):
    MK = M * K
    HALF = MK // 2
    CTOT = C + 6
    NSLAB = 2 * B
    SPS = (NC * NS) // NSLAB
    CPS = (CTOT + SPS - 1) // SPS
    CHUNK = 8192
    NCHUNK = HALF // CHUNK
    NGRP = CHUNK // L
    NBUF = 4
    mesh = plsc.VectorSubcoreMesh(core_axis_name="c", subcore_axis_name="s")

    @functools.partial(
        pl.kernel,
        out_type=(
            jax.ShapeDtypeStruct((B * (C + 3) * MK,), jnp.float32),
            jax.ShapeDtypeStruct((B * 3 * MK,), jnp.float32),
        ),
        mesh=mesh,
        compiler_params=pltpu.CompilerParams(needs_layout_passes=False),
        scratch_types=[
            pltpu.VMEM((HALF,), jnp.int32),
            pltpu.VMEM((N,), jnp.float32),
            pltpu.VMEM((M,), jnp.float32),
            [pltpu.VMEM((CHUNK,), jnp.float32) for _ in range(NBUF)],
            [pltpu.SemaphoreType.DMA for _ in range(NBUF)],
            pltpu.SemaphoreType.DMA,
        ],
    )
    def gather(idx_hbm, feat_hbm, xyzg_hbm, xyz_hbm, nxg_hbm, nx_hbm,
               ofeat_hbm, ogxyz_hbm, ibuf, tbl, nxtbl, obufs, osems, sem0):
        cid = lax.axis_index("c")
        sid = lax.axis_index("s")
        wid = sid * NC + cid
        slab = wid // SPS
        lane = wid % SPS
        b = slab // 2
        h = slab % 2
        slab_off = h * HALF
        lanes0 = lax.iota(jnp.int32, L)

        pltpu.sync_copy(idx_hbm.at[pl.ds(b * MK + slab_off, HALF)], ibuf)

        def do_channel(ci, _):
            c_glob = lane * CPS + ci

            @pl.when(c_glob < CTOT)
            def _():
                is_feat = c_glob < C
                is_fxyz = jnp.logical_and(c_glob >= C, c_glob < C + 3)
                is_gxyz = c_glob >= C + 3
                cf = jnp.minimum(c_glob, C - 1)
                cx = jnp.clip(c_glob - C, 0, 2)
                cg = jnp.clip(c_glob - (C + 3), 0, 2)
                oc = jnp.minimum(c_glob, C + 2)

                @pl.when(is_feat)
                def _():
                    pltpu.sync_copy(feat_hbm.at[b, cf], tbl)

                @pl.when(is_fxyz)
                def _():
                    pltpu.sync_copy(xyzg_hbm.at[b, cx], tbl)
                    pltpu.sync_copy(nxg_hbm.at[b, cx], nxtbl)

                @pl.when(is_gxyz)
                def _():
                    pltpu.sync_copy(xyz_hbm.at[b, cg], tbl)
                    pltpu.sync_copy(nx_hbm.at[b, cg], nxtbl)

                def fill(obuf, ch):
                    base = ch * CHUNK

                    def grp(i):
                        o = i * L
                        iv = ibuf[pl.ds(base + o, L)]
                        g = plsc.load_gather(tbl, [iv])
                        obuf[pl.ds(o, L)] = g

                    def grp_sub(i):
                        o = i * L
                        iv = ibuf[pl.ds(base + o, L)]
                        g = plsc.load_gather(tbl, [iv])
                        mv = (slab_off + base + o + lanes0) // K
                        nxv = plsc.load_gather(nxtbl, [mv])
                        obuf[pl.ds(o, L)] = g - nxv

                    @pl.when(is_feat)
                    def _():
                        plsc.parallel_loop(0, NGRP, 1, unroll=8)(grp)

                    @pl.when(jnp.logical_not(is_feat))
                    def _():
                        plsc.parallel_loop(0, NGRP, 1, unroll=8)(grp_sub)

                def flush(obuf, ch, sem):
                    dst_off = slab_off + ch * CHUNK
                    feat_at = (b * (C + 3) + oc) * MK + dst_off
                    gxyz_at = (b * 3 + cg) * MK + dst_off

                    @pl.when(jnp.logical_not(is_gxyz))
                    def _():
                        pltpu.async_copy(
                            obuf, ofeat_hbm.at[pl.ds(feat_at, CHUNK)], sem)

                    @pl.when(is_gxyz)
                    def _():
                        pltpu.async_copy(
                            obuf, ogxyz_hbm.at[pl.ds(gxyz_at, CHUNK)], sem)

                def wait_flush(obuf, ch, sem):
                    dst_off = slab_off + ch * CHUNK
                    feat_at = (b * (C + 3) + oc) * MK + dst_off
                    gxyz_at = (b * 3 + cg) * MK + dst_off

                    @pl.when(jnp.logical_not(is_gxyz))
                    def _():
                        pltpu.make_async_copy(
                            obuf, ofeat_hbm.at[pl.ds(feat_at, CHUNK)],
                            sem).wait()

                    @pl.when(is_gxyz)
                    def _():
                        pltpu.make_async_copy(
                            obuf, ogxyz_hbm.at[pl.ds(gxyz_at, CHUNK)],
                            sem).wait()

                def ring(j, _):
                    for s in range(NBUF):
                        ch = j * NBUF + s

                        @pl.when(j > 0)
                        def _():
                            wait_flush(obufs[s], ch - NBUF, osems[s])

                        fill(obufs[s], ch)
                        flush(obufs[s], ch, osems[s])
                    return 0

                lax.fori_loop(0, NCHUNK // NBUF, ring, 0)
                for s in range(NBUF):
                    wait_flush(obufs[s], NCHUNK - NBUF + s, osems[s])

            return 0

        lax.fori_loop(0, CPS, do_channel, 0)

    return gather


def kernel(new_xyz, xyz, feature, use_xyz):
    B, _, M = new_xyz.shape
    C = feature.shape[1]
    N = xyz.shape[2]
    cand = _ball_query_cand(new_xyz, xyz)
    select = _make_select(B, M, N)
    idx = select(cand)
    gate = (jnp.asarray(use_xyz) != 0).astype(jnp.float32)
    xyz_g = xyz * gate
    nx_g = new_xyz * gate
    gather = _make_gather(B, C, M, N)
    ofeat, ogxyz = gather(idx, feature, xyz_g, xyz, nx_g, new_xyz)
    group_feature = ofeat.reshape(B, C + 3, M, K)
    group_xyz = ogxyz.reshape(B, 3, M, K)
    return (group_feature, group_xyz)


# speedup vs baseline: 1.0407x; 1.0407x over previous
import functools

import jax
import jax.numpy as jnp
from jax import lax
from jax.experimental import pallas as pl
from jax.experimental.pallas import tpu as pltpu
from jax.experimental.pallas import tpu_sc as plsc

RADIUS = 0.2
K = 64
NC = 2
NS = 16
L = 16


def _cand_body(nxyz_t_ref, xyz_ref, cand_ref):
    nx = nxyz_t_ref[0]
    p = xyz_ref[0]
    qx = nx[:, 0:1]
    qy = nx[:, 1:2]
    qz = nx[:, 2:3]
    px = p[0:1, :]
    py = p[1:2, :]
    pz = p[2:3, :]
    cross = jax.lax.dot_general(nx, p, (((1,), (0,)), ((), ())),
                                preferred_element_type=jnp.float32)
    q2 = (qx * qx + qy * qy) + qz * qz
    p2 = (px * px + py * py) + pz * pz
    d2 = (q2 + p2) - 2.0 * cross
    iota = jax.lax.broadcasted_iota(jnp.int32, d2.shape, 1)
    r2 = jnp.float32(RADIUS * RADIUS)
    cand_ref[0] = jnp.where(d2 < r2, iota, -1)


def _ball_query_cand(new_xyz, xyz):
    B, _, M = new_xyz.shape
    N = xyz.shape[2]
    Mb = 256
    nxyz_t = jnp.transpose(new_xyz, (0, 2, 1))
    grid = (B, M // Mb)
    return pl.pallas_call(
        _cand_body,
        grid=grid,
        in_specs=[
            pl.BlockSpec((1, Mb, 3), lambda b, i: (b, i, 0)),
            pl.BlockSpec((1, 3, N), lambda b, i: (b, 0, 0)),
        ],
        out_specs=pl.BlockSpec((1, Mb, N), lambda b, i: (b, i, 0)),
        out_shape=jax.ShapeDtypeStruct((B, M, N), jnp.int32),
    )(nxyz_t, xyz)


def _make_select(B, M, n):
    rows = B * M
    ngroups = n // L
    rpw = rows // (NC * NS)
    npairs = rpw // 2
    UNROLL = 8
    mesh = plsc.VectorSubcoreMesh(core_axis_name="c", subcore_axis_name="s")

    @functools.partial(
        pl.kernel,
        out_type=jax.ShapeDtypeStruct((rows * K,), jnp.int32),
        mesh=mesh,
        compiler_params=pltpu.CompilerParams(
            needs_layout_passes=False, use_tc_tiling_on_sc=True),
        scratch_types=[
            pltpu.VMEM((n,), jnp.int32),
            pltpu.VMEM((n,), jnp.int32),
            pltpu.VMEM((n + L,), jnp.int32),
            pltpu.VMEM((rpw * K,), jnp.int32),
            pltpu.SemaphoreType.DMA,
            pltpu.SemaphoreType.DMA,
        ],
    )
    def select(cand_hbm, idx_hbm, cbuf0, cbuf1, rowbuf, outbuf, sem0, sem1):
        cid = lax.axis_index("c")
        sid = lax.axis_index("s")
        wid = sid * NC + cid
        base = wid * rpw

        zeros16 = jnp.zeros((L,), jnp.int32)

        def start_row_copy(r, cbuf, sem):
            pltpu.async_copy(cand_hbm.at[r // M, r % M], cbuf, sem)

        def wait_row_copy(r, cbuf, sem):
            pltpu.make_async_copy(cand_hbm.at[r // M, r % M], cbuf, sem).wait()

        BLK = 32

        def scan_row(cbuf, r_local):
            def body(w, ptrv):
                v = cbuf[pl.ds(w * L, L)]
                msk = v >= 0
                cum = plsc.cumsum(msk.astype(jnp.int32))
                pos = ptrv + cum - 1
                mske = jnp.logical_and(msk, pos < K)
                plsc.store_scatter(rowbuf, [pos], v, mask=mske)
                return ptrv + plsc.all_reduce_population_count(msk)

            def blk(bi, ptrv):
                return lax.cond(
                    ptrv[0] >= K,
                    lambda p: p,
                    lambda p: plsc.parallel_loop(
                        bi * BLK, (bi + 1) * BLK, 1, unroll=UNROLL, carry=p)(body),
                    ptrv)

            ptrv = lax.fori_loop(0, ngroups // BLK, blk, zeros16)
            v0 = rowbuf[pl.ds(0, L)]
            lanes0 = lax.iota(jnp.int32, L)
            firstv = plsc.cummax(jnp.where(lanes0 == 0, v0, jnp.int32(-2147483648)))
            fvec = jnp.where(ptrv > 0, firstv, 0)
            obase = r_local * K
            for g in range(K // L):
                cur = rowbuf[pl.ds(g * L, L)]
                lanes = lax.iota(jnp.int32, L) + (g * L)
                outbuf[pl.ds(obase + g * L, L)] = jnp.where(lanes < ptrv, cur, fvec)

        start_row_copy(base, cbuf0, sem0)

        def pair(j, _):
            r0 = base + 2 * j
            start_row_copy(r0 + 1, cbuf1, sem1)
            wait_row_copy(r0, cbuf0, sem0)
            scan_row(cbuf0, 2 * j)

            @pl.when(j < npairs - 1)
            def _():
                start_row_copy(r0 + 2, cbuf0, sem0)

            wait_row_copy(r0 + 1, cbuf1, sem1)
            scan_row(cbuf1, 2 * j + 1)
            return 0

        lax.fori_loop(0, npairs, pair, 0)
        pltpu.sync_copy(outbuf, idx_hbm.at[pl.ds(base * K, rpw * K)])

    return select


def _make_gather(B, C, M, N):
    MK = M * K
    HALF = MK // 2
    CTOT = C + 6
    NSLAB = 2 * B
    SPS = (NC * NS) // NSLAB
    CPS = (CTOT + SPS - 1) // SPS
    CHUNK = 8192
    NCHUNK = HALF // CHUNK
    NGRP = CHUNK // L
    NBUF = 4
    mesh = plsc.VectorSubcoreMesh(core_axis_name="c", subcore_axis_name="s")

    @functools.partial(
        pl.kernel,
        out_type=(
            jax.ShapeDtypeStruct((B * (C + 3) * MK,), jnp.float32),
            jax.ShapeDtypeStruct((B * 3 * MK,), jnp.float32),
        ),
        mesh=mesh,
        compiler_params=pltpu.CompilerParams(needs_layout_passes=False),
        scratch_types=[
            pltpu.VMEM((HALF,), jnp.int32),
            pltpu.VMEM((N,), jnp.float32),
            pltpu.VMEM((M,), jnp.float32),
            [pltpu.VMEM((CHUNK,), jnp.float32) for _ in range(NBUF)],
            [pltpu.SemaphoreType.DMA for _ in range(NBUF)],
            pltpu.SemaphoreType.DMA,
        ],
    )
    def gather(idx_hbm, feat_hbm, xyzg_hbm, xyz_hbm, nxg_hbm, nx_hbm,
               ofeat_hbm, ogxyz_hbm, ibuf, tbl, nxtbl, obufs, osems, sem0):
        cid = lax.axis_index("c")
        sid = lax.axis_index("s")
        wid = sid * NC + cid
        slab = wid // SPS
        lane = wid % SPS
        b = slab // 2
        h = slab % 2
        slab_off = h * HALF
        lanes0 = lax.iota(jnp.int32, L)

        pltpu.sync_copy(idx_hbm.at[pl.ds(b * MK + slab_off, HALF)], ibuf)

        def do_channel(ci, _):
            c_glob = lane * CPS + ci

            @pl.when(c_glob < CTOT)
            def _():
                is_feat = c_glob < C
                is_fxyz = jnp.logical_and(c_glob >= C, c_glob < C + 3)
                is_gxyz = c_glob >= C + 3
                cf = jnp.minimum(c_glob, C - 1)
                cx = jnp.clip(c_glob - C, 0, 2)
                cg = jnp.clip(c_glob - (C + 3), 0, 2)
                oc = jnp.minimum(c_glob, C + 2)

                @pl.when(is_feat)
                def _():
                    pltpu.sync_copy(feat_hbm.at[b, cf], tbl)

                @pl.when(is_fxyz)
                def _():
                    pltpu.sync_copy(xyzg_hbm.at[b, cx], tbl)
                    pltpu.sync_copy(nxg_hbm.at[b, cx], nxtbl)

                @pl.when(is_gxyz)
                def _():
                    pltpu.sync_copy(xyz_hbm.at[b, cg], tbl)
                    pltpu.sync_copy(nx_hbm.at[b, cg], nxtbl)

                def fill(obuf, ch):
                    base = ch * CHUNK

                    def grp(i):
                        o = i * L
                        iv = ibuf[pl.ds(base + o, L)]
                        g = plsc.load_gather(tbl, [iv])
                        obuf[pl.ds(o, L)] = g

                    def grp_sub(i):
                        o = i * L
                        iv = ibuf[pl.ds(base + o, L)]
                        g = plsc.load_gather(tbl, [iv])
                        mv = (slab_off + base + o + lanes0) // K
                        nxv = plsc.load_gather(nxtbl, [mv])
                        obuf[pl.ds(o, L)] = g - nxv

                    @pl.when(is_feat)
                    def _():
                        plsc.parallel_loop(0, NGRP, 1, unroll=8)(grp)

                    @pl.when(jnp.logical_not(is_feat))
                    def _():
                        plsc.parallel_loop(0, NGRP, 1, unroll=8)(grp_sub)

                def flush(obuf, ch, sem):
                    dst_off = slab_off + ch * CHUNK
                    feat_at = (b * (C + 3) + oc) * MK + dst_off
                    gxyz_at = (b * 3 + cg) * MK + dst_off

                    @pl.when(jnp.logical_not(is_gxyz))
                    def _():
                        pltpu.async_copy(
                            obuf, ofeat_hbm.at[pl.ds(feat_at, CHUNK)], sem)

                    @pl.when(is_gxyz)
                    def _():
                        pltpu.async_copy(
                            obuf, ogxyz_hbm.at[pl.ds(gxyz_at, CHUNK)], sem)

                def wait_flush(obuf, ch, sem):
                    dst_off = slab_off + ch * CHUNK
                    feat_at = (b * (C + 3) + oc) * MK + dst_off
                    gxyz_at = (b * 3 + cg) * MK + dst_off

                    @pl.when(jnp.logical_not(is_gxyz))
                    def _():
                        pltpu.make_async_copy(
                            obuf, ofeat_hbm.at[pl.ds(feat_at, CHUNK)],
                            sem).wait()

                    @pl.when(is_gxyz)
                    def _():
                        pltpu.make_async_copy(
                            obuf, ogxyz_hbm.at[pl.ds(gxyz_at, CHUNK)],
                            sem).wait()

                def ring(j, _):
                    for s in range(NBUF):
                        ch = j * NBUF + s

                        @pl.when(j > 0)
                        def _():
                            wait_flush(obufs[s], ch - NBUF, osems[s])

                        fill(obufs[s], ch)
                        flush(obufs[s], ch, osems[s])
                    return 0

                lax.fori_loop(0, NCHUNK // NBUF, ring, 0)
                for s in range(NBUF):
                    wait_flush(obufs[s], NCHUNK - NBUF + s, osems[s])

            return 0

        lax.fori_loop(0, CPS, do_channel, 0)

    return gather


def kernel(new_xyz, xyz, feature, use_xyz):
    B, _, M = new_xyz.shape
    C = feature.shape[1]
    N = xyz.shape[2]
    cand = _ball_query_cand(new_xyz, xyz)
    select = _make_select(B, M, N)
    idx = select(cand)
    gate = (jnp.asarray(use_xyz) != 0).astype(jnp.float32)
    xyz_g = xyz * gate
    nx_g = new_xyz * gate
    gather = _make_gather(B, C, M, N)
    ofeat, ogxyz = gather(idx, feature, xyz_g, xyz, nx_g, new_xyz)
    group_feature = ofeat.reshape(B, C + 3, M, K)
    group_xyz = ogxyz.reshape(B, 3, M, K)
    return (group_feature, group_xyz)
